# Initial kernel scaffold; baseline (speedup 1.0000x reference)
import jax
import jax.numpy as jnp
from jax.experimental import pallas as pl


def kernel(x, breakpoints, intercepts, signs, exps):
    raise NotImplementedError("write your pallas kernel here")


# speedup vs baseline: 3.9442x; 3.9442x over previous
import functools

import jax
import jax.numpy as jnp
from jax import lax
from jax.experimental import pallas as pl
from jax.experimental.pallas import tpu as pltpu
from jax.experimental.pallas import tpu_sc as plsc

_SCALE = 65536.0
_N = 16777216
_NC, _NS, _L = 2, 16, 16
_NW = _NC * _NS
_PER_W = _N // _NW
_C = 16384
_NCHUNK = _PER_W // _C
_NVEC = _C // _L
_UNROLL = 8


def _compute_chunk(xs, ys, bp_v, b_v, la_v, ra_v, s_v):
    @plsc.parallel_loop(0, _NVEC, 1, unroll=_UNROLL)
    def _(v):
        sl = pl.ds(v * _L, _L)
        xq = (xs[sl] * _SCALE).astype(jnp.int32)
        pos = jnp.zeros((_L,), jnp.int32)
        for step in (8, 4, 2, 1):
            g = plsc.load_gather(bp_v, [pos + (step - 1)])
            pos = jnp.where(g <= xq, pos + step, pos)
        b = plsc.load_gather(b_v, [pos])
        la = plsc.load_gather(la_v, [pos])
        ra = plsc.load_gather(ra_v, [pos])
        sg = plsc.load_gather(s_v, [pos])
        t = jnp.right_shift(jnp.left_shift(xq, la), ra)
        y = b + sg * t
        ys[sl] = y.astype(jnp.float32) * (1.0 / _SCALE)


def _body(x_hbm, bp_hbm, b_hbm, la_hbm, ra_hbm, s_hbm, out_hbm,
          xbuf0, xbuf1, ybuf0, ybuf1, bp_v, b_v, la_v, ra_v, s_v,
          sin0, sin1, sout0, sout1):
    xbuf = (xbuf0, xbuf1)
    ybuf = (ybuf0, ybuf1)
    wid = lax.axis_index("s") * _NC + lax.axis_index("c")
    base = wid * _PER_W
    pltpu.sync_copy(bp_hbm, bp_v)
    pltpu.sync_copy(b_hbm, b_v)
    pltpu.sync_copy(la_hbm, la_v)
    pltpu.sync_copy(ra_hbm, ra_v)
    pltpu.sync_copy(s_hbm, s_v)

    sin = (sin0, sin1)
    sout = (sout0, sout1)
    pltpu.async_copy(x_hbm.at[pl.ds(base, _C)], xbuf[0], sin0)
    pltpu.async_copy(x_hbm.at[pl.ds(base + _C, _C)], xbuf[1], sin1)

    @pl.loop(0, _NCHUNK, step=2)
    def _(ci0):
        for slot in range(2):
            ci = ci0 + slot
            off = base + ci * _C
            pltpu.make_async_copy(
                x_hbm.at[pl.ds(off, _C)], xbuf[slot], sin[slot]).wait()
            @pl.when(ci >= 2)
            def _():
                pltpu.make_async_copy(
                    ybuf[slot], out_hbm.at[pl.ds(off - 2 * _C, _C)],
                    sout[slot]).wait()
            _compute_chunk(xbuf[slot], ybuf[slot],
                           bp_v, b_v, la_v, ra_v, s_v)
            pltpu.async_copy(
                ybuf[slot], out_hbm.at[pl.ds(off, _C)], sout[slot])
            @pl.when(ci + 2 < _NCHUNK)
            def _():
                pltpu.async_copy(
                    x_hbm.at[pl.ds(off + 2 * _C, _C)], xbuf[slot],
                    sin[slot])

    for slot in range(2):
        off = base + (_NCHUNK - 2 + slot) * _C
        pltpu.make_async_copy(
            ybuf[slot], out_hbm.at[pl.ds(off, _C)], sout[slot]).wait()


def kernel(x, breakpoints, intercepts, signs, exps):
    bp16 = jnp.concatenate(
        [breakpoints.astype(jnp.int32), jnp.zeros((1,), jnp.int32)])
    la = jnp.maximum(exps, 0).astype(jnp.int32)
    ra = jnp.maximum(-exps, 0).astype(jnp.int32)

    mesh = plsc.VectorSubcoreMesh(core_axis_name="c", subcore_axis_name="s")
    run = functools.partial(
        pl.kernel,
        out_type=jax.ShapeDtypeStruct((_N,), jnp.float32),
        mesh=mesh,
        compiler_params=pltpu.CompilerParams(needs_layout_passes=False),
        scratch_types=[
            pltpu.VMEM((_C,), jnp.float32),
            pltpu.VMEM((_C,), jnp.float32),
            pltpu.VMEM((_C,), jnp.float32),
            pltpu.VMEM((_C,), jnp.float32),
            pltpu.VMEM((_L,), jnp.int32),
            pltpu.VMEM((_L,), jnp.int32),
            pltpu.VMEM((_L,), jnp.int32),
            pltpu.VMEM((_L,), jnp.int32),
            pltpu.VMEM((_L,), jnp.int32),
            pltpu.SemaphoreType.DMA,
            pltpu.SemaphoreType.DMA,
            pltpu.SemaphoreType.DMA,
            pltpu.SemaphoreType.DMA,
        ],
    )(_body)
    return run(x, bp16, intercepts.astype(jnp.int32), la, ra,
               signs.astype(jnp.int32))
